# Initial kernel scaffold; baseline (speedup 1.0000x reference)
import jax
import jax.numpy as jnp
from jax.experimental import pallas as pl


def kernel(x, edge_index, W, att_src, att_dst, bias, W_lin, b_lin):
    raise NotImplementedError("write your pallas kernel here")


# speedup vs baseline: 56.5045x; 56.5045x over previous
import functools

import jax
import jax.numpy as jnp
from jax import lax
from jax.experimental import pallas as pl
from jax.experimental.pallas import tpu as pltpu
from jax.experimental.pallas import tpu_sc as plsc

N = 10000
E = 320000
IN_CH = 128
HEADS = 8
HID = 16
F = HEADS * HID
OUT_CH = 64
AW = 16

NC = 2
NS = 16
NW = NC * NS
EPW = E // NW
CHUNK = 80
NCHUNK = EPW // CHUNK
NPAD = 10240
RPT = NPAD // NS
ZR = 128

TCB = 1000


def _proj_body(x_ref, w_ref, as_ref, ad_ref, h_ref, s_ref, d_ref):
    h = jnp.dot(x_ref[...], w_ref[...], preferred_element_type=jnp.float32)
    h_ref[...] = h
    s_ref[...] = jnp.dot(h, as_ref[...], preferred_element_type=jnp.float32)
    d_ref[...] = jnp.dot(h, ad_ref[...], preferred_element_type=jnp.float32)


def _proj(x, W, a_s, a_d):
    return pl.pallas_call(
        _proj_body,
        grid=(N // TCB,),
        in_specs=[
            pl.BlockSpec((TCB, IN_CH), lambda i: (i, 0)),
            pl.BlockSpec((IN_CH, F), lambda i: (0, 0)),
            pl.BlockSpec((F, AW), lambda i: (0, 0)),
            pl.BlockSpec((F, AW), lambda i: (0, 0)),
        ],
        out_specs=[
            pl.BlockSpec((TCB, F), lambda i: (i, 0)),
            pl.BlockSpec((TCB, AW), lambda i: (i, 0)),
            pl.BlockSpec((TCB, AW), lambda i: (i, 0)),
        ],
        out_shape=[
            jax.ShapeDtypeStruct((N, F), jnp.float32),
            jax.ShapeDtypeStruct((N, AW), jnp.float32),
            jax.ShapeDtypeStruct((N, AW), jnp.float32),
        ],
    )(x, W, a_s, a_d)


def _edge_body(h_hbm, asrc_hbm, adst_hbm, src_hbm, dst_hbm,
               acc_out, den_out,
               sidx, didx, asb, adb, exb, hb, zb, zbd,
               acc_s, den_s, sem):
    cid = lax.axis_index("c")
    sid = lax.axis_index("s")
    wid = cid * NS + sid

    def _zero_zb(i, _):
        for j in range(F // 16):
            zb[i, pl.ds(j * 16, 16)] = jnp.zeros((16,), jnp.float32)
        zbd[i] = jnp.zeros((16,), jnp.float32)
        return _
    lax.fori_loop(0, ZR, _zero_zb, None)
    for k in range(RPT // ZR):
        pltpu.sync_copy(zb, acc_s.at[pl.ds(sid * RPT + k * ZR, ZR)])
        pltpu.sync_copy(zbd, den_s.at[pl.ds(sid * RPT + k * ZR, ZR)])
    plsc.subcore_barrier()

    def _chunk(c, _):
        base = wid * EPW + c * CHUNK
        pltpu.sync_copy(src_hbm.at[pl.ds(base, CHUNK)], sidx)
        pltpu.sync_copy(dst_hbm.at[pl.ds(base, CHUNK)], didx)
        pltpu.async_copy(asrc_hbm.at[sidx], asb, sem).wait()
        pltpu.async_copy(adst_hbm.at[didx], adb, sem).wait()
        pltpu.async_copy(h_hbm.at[sidx], hb, sem).wait()

        def _ex(e, _):
            v = asb[e] + adb[e]
            v = jnp.maximum(v, 0.2 * v)
            exb[e] = jnp.exp(v)
            return _
        lax.fori_loop(0, CHUNK, _ex, None)
        pltpu.sync_copy(exb, den_s.at[didx], add=True)

        def _scale(e, _):
            exrow = exb[e]
            for hh in range(HEADS):
                s = exrow[hh]
                hb[e, pl.ds(hh * HID, HID)] = hb[e, pl.ds(hh * HID, HID)] * s
            return _
        lax.fori_loop(0, CHUNK, _scale, None)
        pltpu.sync_copy(hb, acc_s.at[didx], add=True)
        return _
    lax.fori_loop(0, NCHUNK, _chunk, None)

    plsc.subcore_barrier()

    r0 = sid * RPT
    pltpu.sync_copy(acc_s.at[pl.ds(r0, RPT)], acc_out.at[cid, pl.ds(r0, RPT)])
    pltpu.sync_copy(den_s.at[pl.ds(r0, RPT)], den_out.at[cid, pl.ds(r0, RPT)])


def _edge(h, asrc_t, adst_t, src, dst):
    mesh = plsc.VectorSubcoreMesh(core_axis_name="c", subcore_axis_name="s")
    f = pl.kernel(
        _edge_body,
        out_type=[
            jax.ShapeDtypeStruct((NC, NPAD, F), jnp.float32),
            jax.ShapeDtypeStruct((NC, NPAD, AW), jnp.float32),
        ],
        mesh=mesh,
        scratch_types=[
            pltpu.VMEM((CHUNK,), jnp.int32),
            pltpu.VMEM((CHUNK,), jnp.int32),
            pltpu.VMEM((CHUNK, AW), jnp.float32),
            pltpu.VMEM((CHUNK, AW), jnp.float32),
            pltpu.VMEM((CHUNK, AW), jnp.float32),
            pltpu.VMEM((CHUNK, F), jnp.float32),
            pltpu.VMEM((ZR, F), jnp.float32),
            pltpu.VMEM((ZR, AW), jnp.float32),
            pltpu.VMEM_SHARED((NPAD, F), jnp.float32),
            pltpu.VMEM_SHARED((NPAD, AW), jnp.float32),
            pltpu.SemaphoreType.DMA,
        ],
        compiler_params=pltpu.CompilerParams(use_tc_tiling_on_sc=False),
    )
    return f(h, asrc_t, adst_t, src, dst)


def _final_body(acc0_ref, acc1_ref, den0_ref, den1_ref, r_ref, bias_ref,
                wl_ref, bl_ref, out_ref):
    acc = acc0_ref[...] + acc1_ref[...]
    den = den0_ref[...] + den1_ref[...]
    rec = 1.0 / (den + 1e-16)
    rec_b = jnp.dot(rec, r_ref[...], preferred_element_type=jnp.float32)
    pre = jnp.maximum(acc * rec_b + bias_ref[...], 0.0)
    out_ref[...] = (
        jnp.dot(pre, wl_ref[...], preferred_element_type=jnp.float32)
        + bl_ref[...]
    )


def _final(acc0, acc1, den0, den1, rmat, bias2, W_lin, bl2):
    return pl.pallas_call(
        _final_body,
        grid=(N // TCB,),
        in_specs=[
            pl.BlockSpec((TCB, F), lambda i: (i, 0)),
            pl.BlockSpec((TCB, F), lambda i: (i, 0)),
            pl.BlockSpec((TCB, AW), lambda i: (i, 0)),
            pl.BlockSpec((TCB, AW), lambda i: (i, 0)),
            pl.BlockSpec((AW, F), lambda i: (0, 0)),
            pl.BlockSpec((1, F), lambda i: (0, 0)),
            pl.BlockSpec((F, OUT_CH), lambda i: (0, 0)),
            pl.BlockSpec((1, OUT_CH), lambda i: (0, 0)),
        ],
        out_specs=pl.BlockSpec((TCB, OUT_CH), lambda i: (i, 0)),
        out_shape=jax.ShapeDtypeStruct((N, OUT_CH), jnp.float32),
    )(acc0, acc1, den0, den1, rmat, bias2, W_lin, bl2)


def kernel(x, edge_index, W, att_src, att_dst, bias, W_lin, b_lin):
    src = edge_index[0].astype(jnp.int32)
    dst = edge_index[1].astype(jnp.int32)

    eye8 = jnp.eye(HEADS, dtype=jnp.float32)
    a_s = (att_src.reshape(HEADS, HID)[:, :, None]
           * eye8[:, None, :]).reshape(F, HEADS)
    a_d = (att_dst.reshape(HEADS, HID)[:, :, None]
           * eye8[:, None, :]).reshape(F, HEADS)
    a_s = jnp.pad(a_s, ((0, 0), (0, AW - HEADS)))
    a_d = jnp.pad(a_d, ((0, 0), (0, AW - HEADS)))

    h, asrc_t, adst_t = _proj(x, W, a_s, a_d)

    acc_c, den_c = _edge(h, asrc_t, adst_t, src, dst)

    rmat = (eye8[:, :, None]
            * jnp.ones((1, 1, HID), jnp.float32)).reshape(HEADS, F)
    rmat = jnp.pad(rmat, ((0, AW - HEADS), (0, 0)))

    return _final(acc_c[0, :N], acc_c[1, :N], den_c[0, :N], den_c[1, :N], rmat,
                  bias.reshape(1, F), W_lin, bl2=b_lin.reshape(1, OUT_CH))


# speedup vs baseline: 61.0852x; 1.0811x over previous
import functools

import jax
import jax.numpy as jnp
from jax import lax
from jax.experimental import pallas as pl
from jax.experimental.pallas import tpu as pltpu
from jax.experimental.pallas import tpu_sc as plsc

N = 10000
E = 320000
IN_CH = 128
HEADS = 8
HID = 16
F = HEADS * HID
OUT_CH = 64
AW = 16

NC = 2
NS = 16
NW = NC * NS
CHUNK = 80
TOTCH = E // CHUNK
NPAD = 10240
RPT = NPAD // NS
ZR = 128

TCB = 1000


def _proj_body(x_ref, w_ref, as_ref, ad_ref, h_ref, s_ref, d_ref):
    h = jnp.dot(x_ref[...], w_ref[...], preferred_element_type=jnp.float32)
    h_ref[...] = h
    s_ref[...] = jnp.dot(h, as_ref[...], preferred_element_type=jnp.float32)
    d_ref[...] = jnp.dot(h, ad_ref[...], preferred_element_type=jnp.float32)


def _proj(x, W, a_s, a_d):
    return pl.pallas_call(
        _proj_body,
        grid=(N // TCB,),
        in_specs=[
            pl.BlockSpec((TCB, IN_CH), lambda i: (i, 0)),
            pl.BlockSpec((IN_CH, F), lambda i: (0, 0)),
            pl.BlockSpec((F, AW), lambda i: (0, 0)),
            pl.BlockSpec((F, AW), lambda i: (0, 0)),
        ],
        out_specs=[
            pl.BlockSpec((TCB, F), lambda i: (i, 0)),
            pl.BlockSpec((TCB, AW), lambda i: (i, 0)),
            pl.BlockSpec((TCB, AW), lambda i: (i, 0)),
        ],
        out_shape=[
            jax.ShapeDtypeStruct((N, F), jnp.float32),
            jax.ShapeDtypeStruct((N, AW), jnp.float32),
            jax.ShapeDtypeStruct((N, AW), jnp.float32),
        ],
    )(x, W, a_s, a_d)


def _edge_body(h_hbm, asrc_hbm, adst_hbm, src_hbm, dst_hbm,
               acc_out, den_out,
               sidx, didx, asb, adb, exb, hb, zb, zbd,
               acc_s, den_s, sem, semi):
    cid = lax.axis_index("c")
    sid = lax.axis_index("s")
    wid = cid * NS + sid

    def _zero_zb(i, _):
        for j in range(F // 16):
            zb[i, pl.ds(j * 16, 16)] = jnp.zeros((16,), jnp.float32)
        zbd[i] = jnp.zeros((16,), jnp.float32)
        return _
    lax.fori_loop(0, ZR, _zero_zb, None)
    for k in range(RPT // ZR):
        pltpu.sync_copy(zb, acc_s.at[pl.ds(sid * RPT + k * ZR, ZR)])
        pltpu.sync_copy(zbd, den_s.at[pl.ds(sid * RPT + k * ZR, ZR)])
    plsc.subcore_barrier()

    nch = (TOTCH // NW) + jnp.where(wid < TOTCH - (TOTCH // NW) * NW, 1, 0)

    def _chunk(c, _):
        base = (wid + NW * c) * CHUNK
        pltpu.sync_copy(src_hbm.at[pl.ds(base, CHUNK)], sidx)
        pltpu.sync_copy(dst_hbm.at[pl.ds(base, CHUNK)], didx)
        g0 = pltpu.async_copy(asrc_hbm.at[sidx], asb, sem)
        g1 = pltpu.async_copy(adst_hbm.at[didx], adb, sem)
        g2 = pltpu.async_copy(h_hbm.at[sidx], hb, sem)
        g0.wait()
        g1.wait()
        g2.wait()

        def _ex(e, _):
            v = asb[e] + adb[e]
            v = jnp.maximum(v, 0.2 * v)
            exb[e] = jnp.exp(v)
            return _
        lax.fori_loop(0, CHUNK, _ex, None, unroll=8)
        pltpu.sync_copy(exb, den_s.at[didx], add=True)

        def _scale(e, _):
            exrow = exb[e]
            for hh in range(HEADS):
                s = exrow[hh]
                hb[e, pl.ds(hh * HID, HID)] = hb[e, pl.ds(hh * HID, HID)] * s
            return _
        lax.fori_loop(0, CHUNK, _scale, None, unroll=2)
        pltpu.sync_copy(hb, acc_s.at[didx], add=True)
        return _
    lax.fori_loop(0, nch, _chunk, None)

    plsc.subcore_barrier()

    r0 = sid * RPT
    pltpu.sync_copy(acc_s.at[pl.ds(r0, RPT)], acc_out.at[cid, pl.ds(r0, RPT)])
    pltpu.sync_copy(den_s.at[pl.ds(r0, RPT)], den_out.at[cid, pl.ds(r0, RPT)])


def _edge(h, asrc_t, adst_t, src, dst):
    mesh = plsc.VectorSubcoreMesh(core_axis_name="c", subcore_axis_name="s")
    f = pl.kernel(
        _edge_body,
        out_type=[
            jax.ShapeDtypeStruct((NC, NPAD, F), jnp.float32),
            jax.ShapeDtypeStruct((NC, NPAD, AW), jnp.float32),
        ],
        mesh=mesh,
        scratch_types=[
            pltpu.VMEM((CHUNK,), jnp.int32),
            pltpu.VMEM((CHUNK,), jnp.int32),
            pltpu.VMEM((CHUNK, AW), jnp.float32),
            pltpu.VMEM((CHUNK, AW), jnp.float32),
            pltpu.VMEM((CHUNK, AW), jnp.float32),
            pltpu.VMEM((CHUNK, F), jnp.float32),
            pltpu.VMEM((ZR, F), jnp.float32),
            pltpu.VMEM((ZR, AW), jnp.float32),
            pltpu.VMEM_SHARED((NPAD, F), jnp.float32),
            pltpu.VMEM_SHARED((NPAD, AW), jnp.float32),
            pltpu.SemaphoreType.DMA,
            pltpu.SemaphoreType.DMA,
        ],
        compiler_params=pltpu.CompilerParams(use_tc_tiling_on_sc=False),
    )
    return f(h, asrc_t, adst_t, src, dst)


def _final_body(acc0_ref, acc1_ref, den0_ref, den1_ref, r_ref, bias_ref,
                wl_ref, bl_ref, out_ref):
    acc = acc0_ref[...] + acc1_ref[...]
    den = den0_ref[...] + den1_ref[...]
    rec = 1.0 / (den + 1e-16)
    rec_b = jnp.dot(rec, r_ref[...], preferred_element_type=jnp.float32)
    pre = jnp.maximum(acc * rec_b + bias_ref[...], 0.0)
    out_ref[...] = (
        jnp.dot(pre, wl_ref[...], preferred_element_type=jnp.float32)
        + bl_ref[...]
    )


def _final(acc0, acc1, den0, den1, rmat, bias2, W_lin, bl2):
    return pl.pallas_call(
        _final_body,
        grid=(N // TCB,),
        in_specs=[
            pl.BlockSpec((TCB, F), lambda i: (i, 0)),
            pl.BlockSpec((TCB, F), lambda i: (i, 0)),
            pl.BlockSpec((TCB, AW), lambda i: (i, 0)),
            pl.BlockSpec((TCB, AW), lambda i: (i, 0)),
            pl.BlockSpec((AW, F), lambda i: (0, 0)),
            pl.BlockSpec((1, F), lambda i: (0, 0)),
            pl.BlockSpec((F, OUT_CH), lambda i: (0, 0)),
            pl.BlockSpec((1, OUT_CH), lambda i: (0, 0)),
        ],
        out_specs=pl.BlockSpec((TCB, OUT_CH), lambda i: (i, 0)),
        out_shape=jax.ShapeDtypeStruct((N, OUT_CH), jnp.float32),
    )(acc0, acc1, den0, den1, rmat, bias2, W_lin, bl2)


def kernel(x, edge_index, W, att_src, att_dst, bias, W_lin, b_lin):
    src = edge_index[0].astype(jnp.int32)
    dst = edge_index[1].astype(jnp.int32)

    eye8 = jnp.eye(HEADS, dtype=jnp.float32)
    a_s = (att_src.reshape(HEADS, HID)[:, :, None]
           * eye8[:, None, :]).reshape(F, HEADS)
    a_d = (att_dst.reshape(HEADS, HID)[:, :, None]
           * eye8[:, None, :]).reshape(F, HEADS)
    a_s = jnp.pad(a_s, ((0, 0), (0, AW - HEADS)))
    a_d = jnp.pad(a_d, ((0, 0), (0, AW - HEADS)))

    h, asrc_t, adst_t = _proj(x, W, a_s, a_d)

    acc_c, den_c = _edge(h, asrc_t, adst_t, src, dst)

    rmat = (eye8[:, :, None]
            * jnp.ones((1, 1, HID), jnp.float32)).reshape(HEADS, F)
    rmat = jnp.pad(rmat, ((0, AW - HEADS), (0, 0)))

    return _final(acc_c[0, :N], acc_c[1, :N], den_c[0, :N], den_c[1, :N], rmat,
                  bias.reshape(1, F), W_lin, bl2=b_lin.reshape(1, OUT_CH))


# speedup vs baseline: 90.2260x; 1.4771x over previous
import functools

import jax
import jax.numpy as jnp
from jax import lax
from jax.experimental import pallas as pl
from jax.experimental.pallas import tpu as pltpu
from jax.experimental.pallas import tpu_sc as plsc

N = 10000
E = 320000
IN_CH = 128
HEADS = 8
HID = 16
F = HEADS * HID
OUT_CH = 64
AW = 16

NC = 2
NS = 16
NW = NC * NS
CHUNK = 80
TOTCH = E // CHUNK
NPAD = 10240
RPT = NPAD // NS
ZR = 128

TCB = 1000


def _proj_body(x_ref, w_ref, as_ref, ad_ref, h_ref, s_ref, d_ref):
    h = jnp.dot(x_ref[...], w_ref[...], preferred_element_type=jnp.float32)
    h_ref[...] = h
    s_ref[...] = jnp.dot(h, as_ref[...], preferred_element_type=jnp.float32)
    d_ref[...] = jnp.dot(h, ad_ref[...], preferred_element_type=jnp.float32)


def _proj(x, W, a_s, a_d):
    return pl.pallas_call(
        _proj_body,
        grid=(N // TCB,),
        in_specs=[
            pl.BlockSpec((TCB, IN_CH), lambda i: (i, 0)),
            pl.BlockSpec((IN_CH, F), lambda i: (0, 0)),
            pl.BlockSpec((F, AW), lambda i: (0, 0)),
            pl.BlockSpec((F, AW), lambda i: (0, 0)),
        ],
        out_specs=[
            pl.BlockSpec((TCB, F), lambda i: (i, 0)),
            pl.BlockSpec((TCB, AW), lambda i: (i, 0)),
            pl.BlockSpec((TCB, AW), lambda i: (i, 0)),
        ],
        out_shape=[
            jax.ShapeDtypeStruct((N, F), jnp.float32),
            jax.ShapeDtypeStruct((N, AW), jnp.float32),
            jax.ShapeDtypeStruct((N, AW), jnp.float32),
        ],
    )(x, W, a_s, a_d)


def _edge_body(h_hbm, asrc_hbm, adst_hbm, src_hbm, dst_hbm,
               acc_out, den_out,
               sidx0, didx0, asb0, adb0, exb0, hb0,
               sidx1, didx1, asb1, adb1, exb1, hb1,
               acc_s, den_s, sem0, sem1):
    cid = lax.axis_index("c")
    sid = lax.axis_index("s")
    wid = cid * NS + sid

    A = (sidx0, didx0, asb0, adb0, exb0, hb0, sem0)
    B = (sidx1, didx1, asb1, adb1, exb1, hb1, sem1)

    def _zero(i, _):
        for j in range(F // 16):
            hb0[i, pl.ds(j * 16, 16)] = jnp.zeros((16,), jnp.float32)
        exb0[i] = jnp.zeros((16,), jnp.float32)
        return _
    lax.fori_loop(0, CHUNK, _zero, None)
    for k in range(RPT // CHUNK):
        pltpu.sync_copy(hb0, acc_s.at[pl.ds(sid * RPT + k * CHUNK, CHUNK)])
        pltpu.sync_copy(exb0, den_s.at[pl.ds(sid * RPT + k * CHUNK, CHUNK)])
    plsc.subcore_barrier()

    def _issue(c, bufs):
        sidx, didx, asb, adb, exb, hb, sem = bufs
        base = (wid + NW * c) * CHUNK
        pltpu.sync_copy(src_hbm.at[pl.ds(base, CHUNK)], sidx)
        pltpu.sync_copy(dst_hbm.at[pl.ds(base, CHUNK)], didx)
        pltpu.async_copy(asrc_hbm.at[sidx], asb, sem)
        pltpu.async_copy(adst_hbm.at[didx], adb, sem)
        pltpu.async_copy(h_hbm.at[sidx], hb, sem)

    def _process(bufs):
        sidx, didx, asb, adb, exb, hb, sem = bufs
        pltpu.make_async_copy(asrc_hbm.at[sidx], asb, sem).wait()
        pltpu.make_async_copy(adst_hbm.at[didx], adb, sem).wait()
        pltpu.make_async_copy(h_hbm.at[sidx], hb, sem).wait()

        def _ex(e, _):
            v = asb[e] + adb[e]
            v = jnp.maximum(v, 0.2 * v)
            exb[e] = jnp.exp(v)
            return _
        lax.fori_loop(0, CHUNK, _ex, None, unroll=8)
        pltpu.sync_copy(exb, den_s.at[didx], add=True)

        def _scale(e, _):
            exrow = exb[e]
            for hh in range(HEADS):
                s = exrow[hh]
                hb[e, pl.ds(hh * HID, HID)] = hb[e, pl.ds(hh * HID, HID)] * s
            return _
        lax.fori_loop(0, CHUNK, _scale, None, unroll=2)
        pltpu.sync_copy(hb, acc_s.at[didx], add=True)

    _issue(0, A)

    def _pair(i, _):
        _issue(2 * i + 1, B)
        _process(A)
        _issue(2 * i + 2, A)
        _process(B)
        return _
    lax.fori_loop(0, (TOTCH // NW - 1) // 2, _pair, None)
    _process(A)

    plsc.subcore_barrier()

    r0 = sid * RPT
    pltpu.sync_copy(acc_s.at[pl.ds(r0, RPT)], acc_out.at[cid, pl.ds(r0, RPT)])
    pltpu.sync_copy(den_s.at[pl.ds(r0, RPT)], den_out.at[cid, pl.ds(r0, RPT)])


def _edge(h, asrc_t, adst_t, src, dst):
    mesh = plsc.VectorSubcoreMesh(core_axis_name="c", subcore_axis_name="s")
    f = pl.kernel(
        _edge_body,
        out_type=[
            jax.ShapeDtypeStruct((NC, NPAD, F), jnp.float32),
            jax.ShapeDtypeStruct((NC, NPAD, AW), jnp.float32),
        ],
        mesh=mesh,
        scratch_types=(
            [pltpu.VMEM((CHUNK,), jnp.int32),
             pltpu.VMEM((CHUNK,), jnp.int32),
             pltpu.VMEM((CHUNK, AW), jnp.float32),
             pltpu.VMEM((CHUNK, AW), jnp.float32),
             pltpu.VMEM((CHUNK, AW), jnp.float32),
             pltpu.VMEM((CHUNK, F), jnp.float32)] * 2
            + [pltpu.VMEM_SHARED((NPAD, F), jnp.float32),
               pltpu.VMEM_SHARED((NPAD, AW), jnp.float32),
               pltpu.SemaphoreType.DMA,
               pltpu.SemaphoreType.DMA]
        ),
        compiler_params=pltpu.CompilerParams(use_tc_tiling_on_sc=False),
    )
    return f(h, asrc_t, adst_t, src, dst)


def _final_body(acc0_ref, acc1_ref, den0_ref, den1_ref, r_ref, bias_ref,
                wl_ref, bl_ref, out_ref):
    acc = acc0_ref[...] + acc1_ref[...]
    den = den0_ref[...] + den1_ref[...]
    rec = 1.0 / (den + 1e-16)
    rec_b = jnp.dot(rec, r_ref[...], preferred_element_type=jnp.float32)
    pre = jnp.maximum(acc * rec_b + bias_ref[...], 0.0)
    out_ref[...] = (
        jnp.dot(pre, wl_ref[...], preferred_element_type=jnp.float32)
        + bl_ref[...]
    )


def _final(acc0, acc1, den0, den1, rmat, bias2, W_lin, bl2):
    return pl.pallas_call(
        _final_body,
        grid=(N // TCB,),
        in_specs=[
            pl.BlockSpec((TCB, F), lambda i: (i, 0)),
            pl.BlockSpec((TCB, F), lambda i: (i, 0)),
            pl.BlockSpec((TCB, AW), lambda i: (i, 0)),
            pl.BlockSpec((TCB, AW), lambda i: (i, 0)),
            pl.BlockSpec((AW, F), lambda i: (0, 0)),
            pl.BlockSpec((1, F), lambda i: (0, 0)),
            pl.BlockSpec((F, OUT_CH), lambda i: (0, 0)),
            pl.BlockSpec((1, OUT_CH), lambda i: (0, 0)),
        ],
        out_specs=pl.BlockSpec((TCB, OUT_CH), lambda i: (i, 0)),
        out_shape=jax.ShapeDtypeStruct((N, OUT_CH), jnp.float32),
    )(acc0, acc1, den0, den1, rmat, bias2, W_lin, bl2)


def kernel(x, edge_index, W, att_src, att_dst, bias, W_lin, b_lin):
    src = edge_index[0].astype(jnp.int32)
    dst = edge_index[1].astype(jnp.int32)

    eye8 = jnp.eye(HEADS, dtype=jnp.float32)
    a_s = (att_src.reshape(HEADS, HID)[:, :, None]
           * eye8[:, None, :]).reshape(F, HEADS)
    a_d = (att_dst.reshape(HEADS, HID)[:, :, None]
           * eye8[:, None, :]).reshape(F, HEADS)
    a_s = jnp.pad(a_s, ((0, 0), (0, AW - HEADS)))
    a_d = jnp.pad(a_d, ((0, 0), (0, AW - HEADS)))

    h, asrc_t, adst_t = _proj(x, W, a_s, a_d)

    acc_c, den_c = _edge(h, asrc_t, adst_t, src, dst)

    rmat = (eye8[:, :, None]
            * jnp.ones((1, 1, HID), jnp.float32)).reshape(HEADS, F)
    rmat = jnp.pad(rmat, ((0, AW - HEADS), (0, 0)))

    return _final(acc_c[0, :N], acc_c[1, :N], den_c[0, :N], den_c[1, :N], rmat,
                  bias.reshape(1, F), W_lin, bl2=b_lin.reshape(1, OUT_CH))
